# Initial kernel scaffold; baseline (speedup 1.0000x reference)
import jax
import jax.numpy as jnp
from jax.experimental import pallas as pl


def kernel(predict, target):
    raise NotImplementedError("write your pallas kernel here")


# speedup vs baseline: 16.0706x; 16.0706x over previous
import jax
import jax.numpy as jnp
from jax.experimental import pallas as pl
from jax.experimental.pallas import tpu as pltpu

_N, _H, _W = 4, 64, 64


def _haus_kernel(pred_ref, targ_ref, out_ref):
    k_i = jax.lax.broadcasted_iota(jnp.int32, (_W, _W), 0)
    k_j = jax.lax.broadcasted_iota(jnp.int32, (_W, _W), 1)
    d2 = ((k_i - k_j) * (k_i - k_j)).astype(jnp.float32)
    inf = jnp.float32(jnp.inf)

    def edt2(src):
        cost = jnp.where(src, jnp.float32(0.0), inf)
        g = jnp.min(cost[:, :, None] + d2[None, :, :], axis=1)
        return jnp.min(g[:, None, :] + d2[:, :, None], axis=0)

    def directed(q, s):
        dd = edt2(s)
        m = jnp.max(jnp.where(q, dd, -inf))
        return jnp.where(jnp.any(q), jnp.sqrt(m) / jnp.float32(_W), jnp.float32(0.0))

    total = jnp.float32(0.0)
    for i in range(_N):
        a = jnp.round(pred_ref[i]) > 0.5
        b = jnp.round(targ_ref[i]) > 0.5
        d_a = directed(a & ~b, b)
        d_b = directed(~a & b, a)
        total = total + jnp.maximum(d_a, d_b)
    out_ref[:, :] = jnp.broadcast_to(total / jnp.float32(_N), (1, 1))


@jax.jit
def kernel(predict, target):
    p = predict.reshape(_N, _H, _W)
    t = target.reshape(_N, _H, _W)
    out = pl.pallas_call(
        _haus_kernel,
        out_shape=jax.ShapeDtypeStruct((1, 1), jnp.float32),
    )(p, t)
    return out[0, 0]
